# baseline (speedup 1.0000x reference)
import jax
import jax.numpy as jnp
from jax.experimental import pallas as pl
from jax.experimental.pallas import tpu as pltpu

D = 1024
N_NEIGH = 26
NPAD = 32
CELL_LANE = 31


def _decode(v):
    q729 = jnp.floor((v + 0.5) * (1.0 / 729.0))
    q27 = jnp.floor((v + 0.5) * (1.0 / 27.0))
    return q729, q27 - 27.0 * q729, v - 27.0 * q27


def _routing(idx_ref):
    f32 = jnp.float32
    idxf = idx_ref[...].astype(f32)
    cellf = idxf[0:1, CELL_LANE:CELL_LANE + 1]
    nx, ny, nz = _decode(idxf)
    cx, cy, cz = _decode(cellf)
    d2 = (nx - cx) ** 2 + (ny - cy) ** 2 + (nz - cz) ** 2
    lane = jax.lax.broadcasted_iota(jnp.int32, (1, NPAD), 1)
    valid = (lane < N_NEIGH).astype(f32)
    lm = (d2 <= 3.5).astype(f32) * valid
    fm = ((d2 > 3.5) & (d2 <= 20.5)).astype(f32) * valid
    dm = (d2 > 20.5).astype(f32) * valid
    return lm, fm, dm, valid


def _body(cs_ref, ns_ref, idx_ref, wf1_ref, wl_ref, wd_ref, wf2_ref,
          wg_ref, bias_ref, out_state_ref, out_ew_ref, acc_ref):
    i = pl.program_id(0)
    f32 = jnp.float32

    def mm(x, w):
        return jnp.dot(x, w, preferred_element_type=f32)

    lm, fm, dm, valid = _routing(idx_ref)
    lc = jnp.sum(lm, axis=1, keepdims=True)
    fc = jnp.sum(fm, axis=1, keepdims=True)
    dc = jnp.sum(dm, axis=1, keepdims=True)
    coeff = jnp.concatenate([
        lm / jnp.maximum(lc, 1.0),
        fm / jnp.maximum(fc, 1.0),
        dm / jnp.maximum(dc, 1.0),
        valid * (1.0 / N_NEIGH),
    ], axis=0)
    means = jnp.dot(coeff, ns_ref[...], preferred_element_type=f32)
    cs = cs_ref[...]

    @pl.when(i == 0)
    def _step0():
        xf = jnp.concatenate([cs, means[1:2, :]], axis=1)
        xl = jnp.concatenate([cs, means[0:1, :]], axis=1)
        xg = jnp.concatenate([cs, means[3:4, :]], axis=1)
        u1 = mm(xf, wf1_ref[...])
        h1 = jnp.tanh(u1 + bias_ref[1:2, :])
        acc_ref[1:2, :] = h1
        acc_ref[0:1, :] = mm(xl, wl_ref[...])
        acc_ref[2:3, :] = mm(cs, wd_ref[0])
        acc_ref[3:4, :] = mm(h1[:, 0:D // 2], wf2_ref[0])
        acc_ref[4:5, 0:3] = mm(xg, wg_ref[...])

    @pl.when(i == 1)
    def _step1():
        ud = acc_ref[2:3, :] + mm(means[2:3, :], wd_ref[0])
        u2 = acc_ref[3:4, :] + mm(acc_ref[1:2, D // 2:D], wf2_ref[0])

        local_out = jnp.tanh(acc_ref[0:1, :] + bias_ref[0:1, :])
        local_out = jnp.where(lc > 0.0, local_out, 0.0)
        func_out = jnp.tanh(u2 + bias_ref[2:3, :]) + cs
        func_out = jnp.where(fc > 0.0, func_out, 0.0)
        dist_out = jnp.tanh(ud + bias_ref[3:4, :])
        dist_out = jnp.where(dc > 0.0, dist_out, 0.0)

        g = acc_ref[4:5, 0:128] + bias_ref[4:5, 0:128]
        lane128 = jax.lax.broadcasted_iota(jnp.int32, (1, 128), 1)
        m3 = lane128 < 3
        gmax = jnp.max(jnp.where(m3, g, -jnp.inf), axis=1, keepdims=True)
        e = jnp.where(m3, jnp.exp(g - gmax), 0.0)
        w = e / jnp.sum(e, axis=1, keepdims=True)
        out_ew_ref[...] = w
        out_state_ref[...] = (w[0:1, 0:1] * local_out
                              + w[0:1, 1:2] * func_out
                              + w[0:1, 2:3] * dist_out)


def kernel(current_state, neighbor_states, cell_idx, neighbor_indices,
           W_local, b_local, W_f1, b_f1, W_f2, b_f2, W_dist, b_dist,
           W_gate, b_gate):
    f32 = jnp.float32
    cs2 = current_state.reshape(1, D)
    ns_p = jnp.pad(neighbor_states, ((0, NPAD - N_NEIGH), (0, 0)))
    idxs = jnp.concatenate([
        jnp.asarray(neighbor_indices, jnp.int32),
        jnp.zeros((CELL_LANE - N_NEIGH,), jnp.int32),
        jnp.asarray(cell_idx, jnp.int32).reshape(1),
    ]).reshape(1, NPAD)
    bias = jnp.concatenate([
        b_local.reshape(1, D), b_f1.reshape(1, D), b_f2.reshape(1, D),
        b_dist.reshape(1, D),
        jnp.pad(b_gate, (0, D - 3)).reshape(1, D),
    ], axis=0)

    const2 = pl.BlockSpec((2 * D, D), lambda i: (0, 0))
    stepped = lambda rows: pl.BlockSpec((1, rows, D),
                                        lambda i: (jnp.minimum(i, 1), 0, 0))

    out_state, out_ew = pl.pallas_call(
        _body,
        grid=(2,),
        in_specs=[
            pl.BlockSpec((1, D), lambda i: (0, 0)),
            pl.BlockSpec((NPAD, D), lambda i: (0, 0)),
            pl.BlockSpec((1, NPAD), lambda i: (0, 0)),
            const2,
            const2,
            stepped(D),
            stepped(D // 2),
            pl.BlockSpec((2 * D, 3), lambda i: (0, 0)),
            pl.BlockSpec((5, D), lambda i: (0, 0)),
        ],
        out_specs=[pl.BlockSpec((1, D), lambda i: (0, 0)),
                   pl.BlockSpec((1, 128), lambda i: (0, 0))],
        out_shape=[jax.ShapeDtypeStruct((1, D), f32),
                   jax.ShapeDtypeStruct((1, 128), f32)],
        scratch_shapes=[pltpu.VMEM((8, D), f32)],
    )(cs2, ns_p, idxs, W_f1, W_local, W_dist.reshape(2, D, D),
      W_f2.reshape(2, D // 2, D), W_gate, bias)

    return out_state.reshape(D), out_ew[0, :3]


# speedup vs baseline: 1.0310x; 1.0310x over previous
import jax
import jax.numpy as jnp
from jax.experimental import pallas as pl
from jax.experimental.pallas import tpu as pltpu

D = 1024
N_NEIGH = 26
NPAD = 32
CELL_LANE = 31


def _decode(v):
    q729 = jnp.floor((v + 0.5) * (1.0 / 729.0))
    q27 = jnp.floor((v + 0.5) * (1.0 / 27.0))
    return q729, q27 - 27.0 * q729, v - 27.0 * q27


def _routing(idx_ref):
    f32 = jnp.float32
    idxf = idx_ref[...].astype(f32)
    cellf = idxf[0:1, CELL_LANE:CELL_LANE + 1]
    nx, ny, nz = _decode(idxf)
    cx, cy, cz = _decode(cellf)
    d2 = (nx - cx) ** 2 + (ny - cy) ** 2 + (nz - cz) ** 2
    lane = jax.lax.broadcasted_iota(jnp.int32, (1, NPAD), 1)
    valid = (lane < N_NEIGH).astype(f32)
    lm = (d2 <= 3.5).astype(f32) * valid
    fm = ((d2 > 3.5) & (d2 <= 20.5)).astype(f32) * valid
    dm = (d2 > 20.5).astype(f32) * valid
    return lm, fm, dm, valid


def _body(cs_ref, ns_ref, idx_ref, wf1_ref, wl_ref, wd_ref, wf2_ref,
          wg_ref, bias_ref, out_state_ref, out_ew_ref, acc_ref):
    i = pl.program_id(0)
    f32 = jnp.float32

    def mm(x, w):
        return jnp.dot(x, w, preferred_element_type=f32)

    lm, fm, dm, valid = _routing(idx_ref)
    lc = jnp.sum(lm, axis=1, keepdims=True)
    fc = jnp.sum(fm, axis=1, keepdims=True)
    dc = jnp.sum(dm, axis=1, keepdims=True)
    cs = cs_ref[...]

    @pl.when(i == 0)
    def _cs_half():
        xg = jnp.concatenate([cs, jnp.sum(ns_ref[...], axis=0, keepdims=True)
                              * (1.0 / N_NEIGH)], axis=1)
        acc_ref[4:5, 0:3] = mm(xg, wg_ref[...])
        acc_ref[0:1, :] = mm(cs, wl_ref[0])
        acc_ref[1:2, :] = mm(cs, wf1_ref[0])
        acc_ref[2:3, :] = mm(cs, wd_ref[0])

    @pl.when(i == 1)
    def _mean_half():
        coeff = jnp.concatenate([
            lm / jnp.maximum(lc, 1.0),
            fm / jnp.maximum(fc, 1.0),
            dm / jnp.maximum(dc, 1.0),
        ], axis=0)
        means = jnp.dot(coeff, ns_ref[...], preferred_element_type=f32)
        acc_ref[0:1, :] += mm(means[0:1, :], wl_ref[0])
        acc_ref[1:2, :] += mm(means[1:2, :], wf1_ref[0])
        acc_ref[2:3, :] += mm(means[2:3, :], wd_ref[0])

    @pl.when(i == 2)
    def _finalize():
        h1 = jnp.tanh(acc_ref[1:2, :] + bias_ref[1:2, :])
        u2 = mm(h1, wf2_ref[...])

        local_out = jnp.tanh(acc_ref[0:1, :] + bias_ref[0:1, :])
        local_out = jnp.where(lc > 0.0, local_out, 0.0)
        func_out = jnp.tanh(u2 + bias_ref[2:3, :]) + cs
        func_out = jnp.where(fc > 0.0, func_out, 0.0)
        dist_out = jnp.tanh(acc_ref[2:3, :] + bias_ref[3:4, :])
        dist_out = jnp.where(dc > 0.0, dist_out, 0.0)

        g = acc_ref[4:5, 0:128] + bias_ref[4:5, 0:128]
        lane128 = jax.lax.broadcasted_iota(jnp.int32, (1, 128), 1)
        m3 = lane128 < 3
        gmax = jnp.max(jnp.where(m3, g, -jnp.inf), axis=1, keepdims=True)
        e = jnp.where(m3, jnp.exp(g - gmax), 0.0)
        w = e / jnp.sum(e, axis=1, keepdims=True)
        out_ew_ref[...] = w
        out_state_ref[...] = (w[0:1, 0:1] * local_out
                              + w[0:1, 1:2] * func_out
                              + w[0:1, 2:3] * dist_out)


def kernel(current_state, neighbor_states, cell_idx, neighbor_indices,
           W_local, b_local, W_f1, b_f1, W_f2, b_f2, W_dist, b_dist,
           W_gate, b_gate):
    f32 = jnp.float32
    cs2 = current_state.reshape(1, D)
    ns_p = jnp.pad(neighbor_states, ((0, NPAD - N_NEIGH), (0, 0)))
    idxs = jnp.concatenate([
        jnp.asarray(neighbor_indices, jnp.int32),
        jnp.zeros((CELL_LANE - N_NEIGH,), jnp.int32),
        jnp.asarray(cell_idx, jnp.int32).reshape(1),
    ]).reshape(1, NPAD)
    bias = jnp.concatenate([
        b_local.reshape(1, D), b_f1.reshape(1, D), b_f2.reshape(1, D),
        b_dist.reshape(1, D),
        jnp.pad(b_gate, (0, D - 3)).reshape(1, D),
    ], axis=0)

    halves = pl.BlockSpec((1, D, D), lambda i: (jnp.minimum(i, 1), 0, 0))

    out_state, out_ew = pl.pallas_call(
        _body,
        grid=(3,),
        in_specs=[
            pl.BlockSpec((1, D), lambda i: (0, 0)),
            pl.BlockSpec((NPAD, D), lambda i: (0, 0)),
            pl.BlockSpec((1, NPAD), lambda i: (0, 0)),
            halves,
            halves,
            halves,
            pl.BlockSpec((D, D), lambda i: (0, 0)),
            pl.BlockSpec((2 * D, 3), lambda i: (0, 0)),
            pl.BlockSpec((5, D), lambda i: (0, 0)),
        ],
        out_specs=[pl.BlockSpec((1, D), lambda i: (0, 0)),
                   pl.BlockSpec((1, 128), lambda i: (0, 0))],
        out_shape=[jax.ShapeDtypeStruct((1, D), f32),
                   jax.ShapeDtypeStruct((1, 128), f32)],
        scratch_shapes=[pltpu.VMEM((8, D), f32)],
    )(cs2, ns_p, idxs, W_f1.reshape(2, D, D), W_local.reshape(2, D, D),
      W_dist.reshape(2, D, D), W_f2, W_gate, bias)

    return out_state.reshape(D), out_ew[0, :3]


# speedup vs baseline: 1.1014x; 1.0683x over previous
import jax
import jax.numpy as jnp
from jax.experimental import pallas as pl
from jax.experimental.pallas import tpu as pltpu

D = 1024
N_NEIGH = 26
NPAD = 32
B = 1024
NBLK = 2 * D // B
GRID = NBLK + 1


def _decode(v):
    q729 = jnp.floor((v + 0.5) * (1.0 / 729.0))
    q27 = jnp.floor((v + 0.5) * (1.0 / 27.0))
    return q729, q27 - 27.0 * q729, v - 27.0 * q27


def _masks(nidx_ref, cell_ref):
    f32 = jnp.float32
    idxf = nidx_ref[...].astype(f32)
    cellf = cell_ref[...].astype(f32)
    nx, ny, nz = _decode(idxf)
    cx, cy, cz = _decode(cellf)
    d2 = (nx - cx) ** 2 + (ny - cy) ** 2 + (nz - cz) ** 2
    lane = jax.lax.broadcasted_iota(jnp.int32, (1, NPAD), 1)
    valid = (lane < N_NEIGH).astype(f32)
    lm = (d2 <= 3.5).astype(f32) * valid
    fm = ((d2 > 3.5) & (d2 <= 20.5)).astype(f32) * valid
    dm = (d2 > 20.5).astype(f32) * valid
    return lm, fm, dm, valid


def _body(cs_ref, ns_ref, nidx_ref, cell_ref, wf1_ref, wl_ref, wd_ref,
          wf2_ref, wg_ref, bl_ref, bf1_ref, bf2_ref, bd_ref, bg_ref,
          out_state_ref, out_ew_ref, acc_ref):
    i = pl.program_id(0)
    f32 = jnp.float32

    def mm(x, w_ref):
        return jnp.dot(x, w_ref[...], preferred_element_type=f32)

    @pl.when(i == 0)
    def _init():
        acc_ref[...] = jnp.zeros_like(acc_ref)
        lm, fm, dm, valid = _masks(nidx_ref, cell_ref)
        lc = jnp.sum(lm, axis=1, keepdims=True)
        fc = jnp.sum(fm, axis=1, keepdims=True)
        dc = jnp.sum(dm, axis=1, keepdims=True)
        coeff = jnp.concatenate([
            lm / jnp.maximum(lc, 1.0),
            fm / jnp.maximum(fc, 1.0),
            dm / jnp.maximum(dc, 1.0),
            valid * (1.0 / N_NEIGH),
        ], axis=0)
        means = jnp.dot(coeff, ns_ref[...], preferred_element_type=f32)
        acc_ref[5:6, :] = means[0:1, :]
        acc_ref[6:7, :] = means[1:2, :]
        acc_ref[7:8, :] = means[2:3, :]
        xg = jnp.concatenate([cs_ref[...], means[3:4, :]], axis=1)
        acc_ref[4:5, 0:3] = mm(xg, wg_ref)

    half = i % (NBLK // 2)

    @pl.when(jnp.logical_and(i < NBLK, i < NBLK // 2))
    def _layer1_cs():
        x = cs_ref[0:1, pl.ds(half * B, B)]
        acc_ref[0:1, :] += mm(x, wl_ref)
        acc_ref[1:2, :] += mm(x, wf1_ref)
        acc_ref[2:3, :] += mm(x, wd_ref)

    @pl.when(jnp.logical_and(i < NBLK, i >= NBLK // 2))
    def _layer1_mean():
        sl = pl.ds(half * B, B)
        acc_ref[0:1, :] += mm(acc_ref[5:6, sl], wl_ref)
        acc_ref[1:2, :] += mm(acc_ref[6:7, sl], wf1_ref)
        acc_ref[2:3, :] += mm(acc_ref[7:8, sl], wd_ref)

    @pl.when(i == NBLK)
    def _finalize():
        lm, fm, dm, _ = _masks(nidx_ref, cell_ref)
        lc = jnp.sum(lm, axis=1, keepdims=True)
        fc = jnp.sum(fm, axis=1, keepdims=True)
        dc = jnp.sum(dm, axis=1, keepdims=True)
        cs = cs_ref[...]
        h1 = jnp.tanh(acc_ref[1:2, :] + bf1_ref[...])
        u2 = mm(h1, wf2_ref)
        local_out = jnp.tanh(acc_ref[0:1, :] + bl_ref[...])
        local_out = jnp.where(lc > 0.0, local_out, 0.0)
        func_out = jnp.tanh(u2 + bf2_ref[...]) + cs
        func_out = jnp.where(fc > 0.0, func_out, 0.0)
        dist_out = jnp.tanh(acc_ref[2:3, :] + bd_ref[...])
        dist_out = jnp.where(dc > 0.0, dist_out, 0.0)

        g = acc_ref[4:5, 0:128] + bg_ref[...]
        lane128 = jax.lax.broadcasted_iota(jnp.int32, (1, 128), 1)
        m3 = lane128 < 3
        gmax = jnp.max(jnp.where(m3, g, -jnp.inf), axis=1, keepdims=True)
        e = jnp.where(m3, jnp.exp(g - gmax), 0.0)
        w = e / jnp.sum(e, axis=1, keepdims=True)
        out_ew_ref[...] = w
        out_state_ref[...] = (w[0:1, 0:1] * local_out
                              + w[0:1, 1:2] * func_out
                              + w[0:1, 2:3] * dist_out)


def kernel(current_state, neighbor_states, cell_idx, neighbor_indices,
           W_local, b_local, W_f1, b_f1, W_f2, b_f2, W_dist, b_dist,
           W_gate, b_gate):
    f32 = jnp.float32
    cs2 = current_state.reshape(1, D)
    ns_p = jnp.pad(neighbor_states, ((0, NPAD - N_NEIGH), (0, 0)))
    nidx = jnp.pad(jnp.asarray(neighbor_indices, jnp.int32),
                   (0, NPAD - N_NEIGH)).reshape(1, NPAD)
    cell = jnp.asarray(cell_idx, jnp.int32).reshape(1, 1)
    bg_p = jnp.pad(b_gate, (0, 128 - 3)).reshape(1, 128)

    full = lambda shape: pl.BlockSpec(shape, lambda i: (0, 0))
    l1_map = lambda i: (jnp.minimum(i, NBLK - 1), 0)

    out_state, out_ew = pl.pallas_call(
        _body,
        grid=(GRID,),
        in_specs=[
            full((1, D)),
            full((NPAD, D)),
            full((1, NPAD)),
            full((1, 1)),
            pl.BlockSpec((B, D), l1_map),
            pl.BlockSpec((B, D), l1_map),
            pl.BlockSpec((B, D), l1_map),
            full((D, D)),
            full((2 * D, 3)),
            full((1, D)),
            full((1, D)),
            full((1, D)),
            full((1, D)),
            full((1, 128)),
        ],
        out_specs=[full((1, D)), full((1, 128))],
        out_shape=[jax.ShapeDtypeStruct((1, D), f32),
                   jax.ShapeDtypeStruct((1, 128), f32)],
        scratch_shapes=[pltpu.VMEM((8, D), f32)],
    )(cs2, ns_p, nidx, cell, W_f1, W_local, W_dist, W_f2, W_gate,
      b_local.reshape(1, D), b_f1.reshape(1, D), b_f2.reshape(1, D),
      b_dist.reshape(1, D), bg_p)

    return out_state.reshape(D), out_ew[0, :3]
